# baseline (speedup 1.0000x reference)
import jax
import jax.numpy as jnp
from jax import lax
from jax.experimental import pallas as pl
from jax.experimental.pallas import tpu as pltpu


_MIX = 0x45D9F3B
_NB = 999999

_ROWS = 200
_COLS = 16384
_R = 8
_CW = 8192
_SPLIT = _COLS // _CW
_C = (_ROWS // _R) * _SPLIT
_D = 50


def _bucket(v):
    u = v.astype(jnp.uint32)
    h = u ^ (u >> 16)
    h = h * jnp.uint32(_MIX)
    h = h ^ (h >> 16)
    h = h * jnp.uint32(_MIX)
    h = h ^ (h >> 16)
    q = h // jnp.uint32(_NB)
    t = (h - q * jnp.uint32(_NB)).astype(jnp.int32)
    return jnp.where(v == 0, 0, t + 1)


def _body(x_hbm, o_hbm, ibuf, obuf, isem, osem):
    def _sl(i):
        g, h = divmod(i, _SPLIT)
        return (pl.ds(g * _R, _R), pl.ds(h * _CW, _CW))

    def in_copy(i, slot):
        return pltpu.make_async_copy(x_hbm.at[_sl(i)], ibuf.at[slot], isem.at[slot])

    def out_copy(i, slot):
        return pltpu.make_async_copy(obuf.at[slot], o_hbm.at[_sl(i)], osem.at[slot])

    for i in range(_D):
        in_copy(i, i).start()
    for i in range(_C):
        slot = i % _D
        in_copy(i, slot).wait()
        if i >= _D:
            out_copy(i - _D, slot).wait()
        obuf[slot] = _bucket(ibuf[slot])
        out_copy(i, slot).start()
        if i + _D < _C:
            in_copy(i + _D, slot).start()
    for i in range(_C - _D, _C):
        out_copy(i, i % _D).wait()


def kernel(x):
    xt = x.T
    out_t = pl.pallas_call(
        _body,
        out_shape=jax.ShapeDtypeStruct((_ROWS, _COLS), jnp.int32),
        in_specs=[pl.BlockSpec(memory_space=pltpu.MemorySpace.HBM)],
        out_specs=pl.BlockSpec(memory_space=pltpu.MemorySpace.HBM),
        scratch_shapes=[
            pltpu.VMEM((_D, _R, _CW), jnp.int32),
            pltpu.VMEM((_D, _R, _CW), jnp.int32),
            pltpu.SemaphoreType.DMA((_D,)),
            pltpu.SemaphoreType.DMA((_D,)),
        ],
    )(xt)
    return out_t.T
